# Initial kernel scaffold; baseline (speedup 1.0000x reference)
import jax
import jax.numpy as jnp
from jax.experimental import pallas as pl


def kernel(x, w1, b1):
    raise NotImplementedError("write your pallas kernel here")


# speedup vs baseline: 1.4041x; 1.4041x over previous
import functools

import jax
import jax.numpy as jnp
from jax.experimental import pallas as pl
from jax.experimental.pallas import tpu as pltpu

_EPS = 1e-4
_BM = 512
_BK = 512


def _ff_body(x_ref, w_ref, b_ref, o_ref):
    k = pl.program_id(1)
    nk = pl.num_programs(1)

    xk = x_ref[:, pl.ds(k * _BK, _BK)].astype(jnp.bfloat16)
    part = jax.lax.dot_general(
        xk, w_ref[...], (((1,), (1,)), ((), ())),
        preferred_element_type=jnp.float32,
    )

    @pl.when(k == 0)
    def _init():
        o_ref[...] = part

    @pl.when(k > 0)
    def _acc():
        o_ref[...] = o_ref[...] + part

    @pl.when(k == nk - 1)
    def _finalize():
        r = jnp.maximum(o_ref[...] + b_ref[...], 0.0) + x_ref[...]
        m = jnp.mean(r, axis=-1, keepdims=True)
        d = r - m
        v = jnp.mean(d * d, axis=-1, keepdims=True)
        o_ref[...] = d / jnp.sqrt(v + _EPS)


@jax.jit
def kernel(x, w1, b1):
    n, f = x.shape
    w_bf = w1.astype(jnp.bfloat16)
    b2d = b1.reshape(1, f)

    grid = (n // _BM, f // _BK)
    return pl.pallas_call(
        _ff_body,
        grid=grid,
        in_specs=[
            pl.BlockSpec((_BM, f), lambda m, k: (m, 0)),
            pl.BlockSpec((f, _BK), lambda m, k: (0, k)),
            pl.BlockSpec((1, f), lambda m, k: (0, 0)),
        ],
        out_specs=pl.BlockSpec((_BM, f), lambda m, k: (m, 0)),
        out_shape=jax.ShapeDtypeStruct((n, f), jnp.float32),
        compiler_params=pltpu.CompilerParams(
            dimension_semantics=("parallel", "arbitrary"),
            vmem_limit_bytes=60 * 1024 * 1024,
        ),
    )(x, w_bf, b2d)


# speedup vs baseline: 2.0232x; 1.4410x over previous
import jax
import jax.numpy as jnp
from jax.experimental import pallas as pl
from jax.experimental.pallas import tpu as pltpu

_EPS = 1e-4
_BM = 256
_NCORES = 2


def _ff_body(x_ref, b_ref, w_hbm, o_ref, w_vmem, sem):
    m = pl.program_id(1)

    @pl.when(m == 0)
    def _load_w():
        cp = pltpu.make_async_copy(w_hbm, w_vmem, sem)
        cp.start()
        cp.wait()

    xb = x_ref[...].astype(jnp.bfloat16)
    acc = jnp.dot(xb, w_vmem[...], preferred_element_type=jnp.float32)
    r = jnp.maximum(acc + b_ref[...], 0.0) + x_ref[...]
    mu = jnp.mean(r, axis=-1, keepdims=True)
    d = r - mu
    v = jnp.mean(d * d, axis=-1, keepdims=True)
    o_ref[...] = d / jnp.sqrt(v + _EPS)


@jax.jit
def kernel(x, w1, b1):
    n, f = x.shape
    w_t = w1.T.astype(jnp.bfloat16)
    b2d = b1.reshape(1, f)

    nm = n // (_BM * _NCORES)
    grid = (_NCORES, nm)
    return pl.pallas_call(
        _ff_body,
        grid=grid,
        in_specs=[
            pl.BlockSpec((_BM, f), lambda c, m: (c * nm + m, 0)),
            pl.BlockSpec((1, f), lambda c, m: (0, 0)),
            pl.BlockSpec(memory_space=pl.ANY),
        ],
        out_specs=pl.BlockSpec((_BM, f), lambda c, m: (c * nm + m, 0)),
        out_shape=jax.ShapeDtypeStruct((n, f), jnp.float32),
        scratch_shapes=[
            pltpu.VMEM((f, f), jnp.bfloat16),
            pltpu.SemaphoreType.DMA,
        ],
        compiler_params=pltpu.CompilerParams(
            dimension_semantics=("parallel", "arbitrary"),
            vmem_limit_bytes=60 * 1024 * 1024,
        ),
    )(x, b2d, w_t)


# speedup vs baseline: 2.0689x; 1.0226x over previous
import jax
import jax.numpy as jnp
from jax.experimental import pallas as pl
from jax.experimental.pallas import tpu as pltpu

_EPS = 1e-4
_BM = 256
_NCORES = 2


def _ff_body(x_ref, b_ref, w_hbm, o_ref, w_vmem, sem):
    m = pl.program_id(1)

    @pl.when(m == 0)
    def _load_w():
        cp = pltpu.make_async_copy(w_hbm, w_vmem, sem)
        cp.start()
        cp.wait()

    xb = x_ref[...].astype(jnp.bfloat16)
    acc = jax.lax.dot_general(
        xb, w_vmem[...], (((1,), (1,)), ((), ())),
        preferred_element_type=jnp.float32,
    )
    r = jnp.maximum(acc + b_ref[...], 0.0) + x_ref[...]
    mu = jnp.mean(r, axis=-1, keepdims=True)
    d = r - mu
    v = jnp.mean(d * d, axis=-1, keepdims=True)
    o_ref[...] = d / jnp.sqrt(v + _EPS)


@jax.jit
def kernel(x, w1, b1):
    n, f = x.shape
    w_t = w1.astype(jnp.bfloat16)
    b2d = b1.reshape(1, f)

    nm = n // (_BM * _NCORES)
    grid = (_NCORES, nm)
    return pl.pallas_call(
        _ff_body,
        grid=grid,
        in_specs=[
            pl.BlockSpec((_BM, f), lambda c, m: (c * nm + m, 0)),
            pl.BlockSpec((1, f), lambda c, m: (0, 0)),
            pl.BlockSpec(memory_space=pl.ANY),
        ],
        out_specs=pl.BlockSpec((_BM, f), lambda c, m: (c * nm + m, 0)),
        out_shape=jax.ShapeDtypeStruct((n, f), jnp.float32),
        scratch_shapes=[
            pltpu.VMEM((f, f), jnp.bfloat16),
            pltpu.SemaphoreType.DMA,
        ],
        compiler_params=pltpu.CompilerParams(
            dimension_semantics=("parallel", "arbitrary"),
            vmem_limit_bytes=60 * 1024 * 1024,
        ),
    )(x, b2d, w_t)
